# baseline (speedup 1.0000x reference)
import jax
import jax.numpy as jnp
from jax.experimental import pallas as pl
from jax.experimental.pallas import tpu as pltpu

CHANNEL = 64
SEG_NUM = 2
SEG_DIM = CHANNEL // SEG_NUM
DICT_SIZE = 512
NUM_STAGES = 3

TOKENS_PER_BLOCK = 2048


def _fused_body(z_ref, wlse_ref, wq_ref, wd_ref, wr_ref, wlh_ref, d_ref, g_ref,
                blse_ref, bq_ref, blh_ref, bd_ref, br_ref, a_ref, numf_ref,
                out_ref):
    f32 = jnp.float32
    latent = z_ref[0]
    restored = jnp.zeros_like(latent)
    iota_f = jax.lax.broadcasted_iota(
        jnp.int32, (latent.shape[0], SEG_NUM * DICT_SIZE), 1).astype(f32)
    for m in range(NUM_STAGES):
        h = jnp.dot(latent, wlse_ref[m], preferred_element_type=f32) + blse_ref[m]
        q = jnp.dot(h, wq_ref[m], preferred_element_type=f32) + bq_ref[m]
        dots = jnp.dot(q, d_ref[m], preferred_element_type=f32)
        dist = a_ref[m] - dots
        quantized = None
        for s in range(SEG_NUM):
            ds = dist[:, s * DICT_SIZE:(s + 1) * DICT_SIZE]
            it = iota_f[:, s * DICT_SIZE:(s + 1) * DICT_SIZE]
            mn = jnp.min(ds, axis=1, keepdims=True)
            cand = jnp.where(ds == mn, it, f32(2 * SEG_NUM * DICT_SIZE))
            idx = jnp.min(cand, axis=1, keepdims=True)
            oh = jnp.where(cand == idx, f32(1.0), f32(0.0))
            t = jnp.dot(oh, g_ref[m, s], preferred_element_type=f32)
            qs = ((t[:, 0:CHANNEL] + t[:, CHANNEL:2 * CHANNEL])
                  + t[:, 2 * CHANNEL:3 * CHANNEL])
            quantized = qs if quantized is None else quantized + qs
        deq = jnp.dot(quantized, wd_ref[m], preferred_element_type=f32) + bd_ref[m]
        restored = restored + jnp.dot(deq, wr_ref[m], preferred_element_type=f32) + br_ref[m]
        latent = jnp.dot(h, wlh_ref[m], preferred_element_type=f32) + blh_ref[m] - deq
    out_ref[0] = restored * numf_ref[0, 0]


@jax.jit
def _run(z, wlse, wq, wd, wr, wlh, d, g, blse, bq, blh, bd, br, a, numf):
    b, hw, _ = z.shape
    grid = (b, hw // TOKENS_PER_BLOCK)
    tok_spec = pl.BlockSpec((1, TOKENS_PER_BLOCK, CHANNEL), lambda i, j: (i, j, 0))
    w_spec = pl.BlockSpec((NUM_STAGES, CHANNEL, CHANNEL), lambda i, j: (0, 0, 0))
    b_spec = pl.BlockSpec((NUM_STAGES, 1, CHANNEL), lambda i, j: (0, 0, 0))
    d_spec = pl.BlockSpec((NUM_STAGES, CHANNEL, SEG_NUM * DICT_SIZE),
                          lambda i, j: (0, 0, 0))
    g_spec = pl.BlockSpec((NUM_STAGES, SEG_NUM, DICT_SIZE, 3 * CHANNEL),
                          lambda i, j: (0, 0, 0, 0))
    a_spec = pl.BlockSpec((NUM_STAGES, 1, SEG_NUM * DICT_SIZE),
                          lambda i, j: (0, 0, 0))
    s_spec = pl.BlockSpec((1, 1), lambda i, j: (0, 0))
    return pl.pallas_call(
        _fused_body,
        grid=grid,
        in_specs=[tok_spec, w_spec, w_spec, w_spec, w_spec, w_spec, d_spec,
                  g_spec, b_spec, b_spec, b_spec, b_spec, b_spec, a_spec,
                  s_spec],
        out_specs=tok_spec,
        out_shape=jax.ShapeDtypeStruct(z.shape, jnp.float32),
        compiler_params=pltpu.CompilerParams(
            dimension_semantics=("parallel", "parallel")),
    )(z, wlse, wq, wd, wr, wlh, d, g, blse, bq, blh, bd, br, a, numf)


def kernel(z,
           W_latentStageEncoder_0, b_latentStageEncoder_0,
           W_quantizationHead_0, b_quantizationHead_0,
           W_latentHead_0, b_latentHead_0,
           W_dequantizationHead_0, b_dequantizationHead_0,
           W_restoreHead_0, b_restoreHead_0,
           codebook_0,
           W_latentStageEncoder_1, b_latentStageEncoder_1,
           W_quantizationHead_1, b_quantizationHead_1,
           W_latentHead_1, b_latentHead_1,
           W_dequantizationHead_1, b_dequantizationHead_1,
           W_restoreHead_1, b_restoreHead_1,
           codebook_1,
           W_latentStageEncoder_2, b_latentStageEncoder_2,
           W_quantizationHead_2, b_quantizationHead_2,
           W_latentHead_2, b_latentHead_2,
           W_dequantizationHead_2, b_dequantizationHead_2,
           W_restoreHead_2, b_restoreHead_2,
           codebook_2,
           num):
    B, HW, C = z.shape
    wq = [W_quantizationHead_0, W_quantizationHead_1, W_quantizationHead_2]
    bq = [b_quantizationHead_0, b_quantizationHead_1, b_quantizationHead_2]
    wd = [W_dequantizationHead_0, W_dequantizationHead_1, W_dequantizationHead_2]
    cbs = [codebook_0, codebook_1, codebook_2]

    d_list, a_list, g_list = [], [], []
    for m in range(NUM_STAGES):
        cb = cbs[m]
        as_, gs = [], []
        dmat = jnp.zeros((CHANNEL, SEG_NUM * DICT_SIZE), dtype=jnp.float32)
        for s in range(SEG_NUM):
            cbt = cb[s].T
            dmat = dmat.at[s * SEG_DIM:(s + 1) * SEG_DIM,
                           s * DICT_SIZE:(s + 1) * DICT_SIZE].set(cbt)
            c2 = jnp.sum(cb[s] * cb[s], axis=1)
            as_.append(0.5 * c2)
            pad = [jnp.zeros((DICT_SIZE, SEG_DIM), jnp.float32)] * SEG_NUM
            pad[s] = cb[s]
            gseg = jnp.concatenate(pad, axis=1)
            hi = gseg.astype(jnp.bfloat16).astype(jnp.float32)
            r = gseg - hi
            mid = r.astype(jnp.bfloat16).astype(jnp.float32)
            lo = r - mid
            gs.append(jnp.concatenate([hi, mid, lo], axis=1))
        d_list.append(dmat)
        a_list.append(jnp.concatenate(as_).reshape(1, SEG_NUM * DICT_SIZE))
        g_list.append(jnp.stack(gs))

    wlse = jnp.stack([W_latentStageEncoder_0, W_latentStageEncoder_1, W_latentStageEncoder_2])
    wqs = jnp.stack(wq)
    bqs = jnp.stack(bq).reshape(NUM_STAGES, 1, C)
    wlh = jnp.stack([W_latentHead_0, W_latentHead_1, W_latentHead_2])
    wr = jnp.stack([W_restoreHead_0, W_restoreHead_1, W_restoreHead_2])
    blse = jnp.stack([b_latentStageEncoder_0, b_latentStageEncoder_1, b_latentStageEncoder_2]).reshape(NUM_STAGES, 1, C)
    blh = jnp.stack([b_latentHead_0, b_latentHead_1, b_latentHead_2]).reshape(NUM_STAGES, 1, C)
    bd = jnp.stack([b_dequantizationHead_0, b_dequantizationHead_1, b_dequantizationHead_2]).reshape(NUM_STAGES, 1, C)
    br = jnp.stack([b_restoreHead_0, b_restoreHead_1, b_restoreHead_2]).reshape(NUM_STAGES, 1, C)
    d = jnp.stack(d_list)
    g = jnp.stack(g_list)
    a = jnp.stack(a_list)
    wds = jnp.stack(wd)
    numf = jnp.asarray(num, jnp.float32).reshape(1, 1)
    return _run(z, wlse, wqs, wds, wr, wlh, d, g, blse, bqs, blh, bd, br, a,
                numf)
